# baseline (speedup 1.0000x reference)
import functools

import jax
import jax.numpy as jnp
from jax import lax
from jax.experimental import pallas as pl
from jax.experimental.pallas import tpu as pltpu
from jax.experimental.pallas import tpu_sc as plsc

E = 8
D = 1024
T = 4096
A = 2 * T
BM = 256
NBLK = A // BM + E
P = NBLK * BM
NC = 2
NS = 16
NW = NC * NS

GROWS = P // NW
GCH = 40
NGCH = GROWS // GCH
GBUF = 3

CTOK = T // NW
CCH = 32
NCCH = CTOK // CCH


def _gate_body(x_ref, wg_ref, comb_ref):
    xblk = x_ref[...]
    logits = lax.dot_general(
        xblk, wg_ref[...], (((1,), (1,)), ((), ())),
        preferred_element_type=jnp.float32)
    z = logits - jnp.max(logits, axis=-1, keepdims=True)
    p = jnp.exp(z)
    p = p / jnp.sum(p, axis=-1, keepdims=True)
    idx = lax.broadcasted_iota(jnp.int32, p.shape, 1)
    m1 = jnp.max(p, axis=-1, keepdims=True)
    i1 = jnp.min(jnp.where(p == m1, idx, E), axis=-1, keepdims=True)
    sel1 = idx == i1
    pm = jnp.where(sel1, -1.0, p)
    m2 = jnp.max(pm, axis=-1, keepdims=True)
    i2 = jnp.min(jnp.where(pm == m2, idx, E), axis=-1, keepdims=True)
    sel2 = idx == i2
    denom = m1 + m2 + 1e-9
    comb_ref[...] = (jnp.where(sel1, m1 / denom, 0.0)
                     + jnp.where(sel2, m2 / denom, 0.0))


def _gate(xt, Wg):
    return pl.pallas_call(
        _gate_body,
        out_shape=jax.ShapeDtypeStruct((T, E), jnp.float32),
    )(xt, Wg)


def _route(comb):
    rows = jnp.arange(T)
    e1 = jnp.argmax(comb, axis=-1).astype(jnp.int32)
    w1 = jnp.max(comb, axis=-1)
    combm = comb.at[rows, e1].set(-1.0)
    e2 = jnp.argmax(combm, axis=-1).astype(jnp.int32)
    w2 = jnp.max(combm, axis=-1)

    ea = jnp.concatenate([e1, e2])
    wa = jnp.concatenate([w1, w2])
    ta = jnp.concatenate([rows, rows]).astype(jnp.int32)

    onehot = (ea[:, None] == jnp.arange(E)[None, :])
    cnt = jnp.sum(onehot, axis=0).astype(jnp.int32)
    pcnt = ((cnt + BM - 1) // BM) * BM
    pend = jnp.cumsum(pcnt)
    pstart = pend - pcnt
    cend = jnp.cumsum(cnt)
    cstart = cend - cnt

    order = jnp.argsort(ea, stable=True)
    inv = jnp.zeros((A,), jnp.int32).at[order].set(jnp.arange(A, dtype=jnp.int32))
    pos = inv - cstart[ea] + pstart[ea]

    src_token = jnp.zeros((P,), jnp.int32).at[pos].set(ta)
    wsorted = jnp.zeros((P,), jnp.float32).at[pos].set(wa)

    blk_base = jnp.arange(NBLK, dtype=jnp.int32) * BM
    bexp = jnp.sum(blk_base[:, None] >= pend[None, :], axis=-1).astype(jnp.int32)
    bexp = jnp.minimum(bexp, E - 1)
    nblk = (pend[-1] // BM).astype(jnp.int32).reshape((1,))
    return src_token, wsorted.reshape(P, 1), bexp, nblk, pos[:T], pos[T:]


def _gather_body(xt_hbm, src_hbm, xg_hbm, *rest):
    ics = rest[:NGCH]
    bufs = rest[NGCH:NGCH + GBUF]
    isem = rest[NGCH + GBUF]
    gsems = rest[NGCH + GBUF + 1:NGCH + 2 * GBUF + 1]
    wsems = rest[NGCH + 2 * GBUF + 1:NGCH + 3 * GBUF + 1]
    wid = lax.axis_index("s") * NC + lax.axis_index("c")
    base = pl.multiple_of(wid * GROWS, GROWS)
    ih = [pltpu.async_copy(src_hbm.at[pl.ds(base + c * GCH, GCH)], ics[c], isem)
          for c in range(NGCH)]
    for h in ih:
        h.wait()

    def start_gather(c):
        return pltpu.async_copy(xt_hbm.at[ics[c]], bufs[c % GBUF],
                                gsems[c % GBUF])

    gh = {}
    wh = {}
    for c in range(min(2, NGCH)):
        gh[c] = start_gather(c)
    for c in range(NGCH):
        gh[c].wait()
        wh[c] = pltpu.async_copy(
            bufs[c % GBUF], xg_hbm.at[pl.ds(base + c * GCH, GCH)],
            wsems[c % GBUF])
        n = c + 2
        if n < NGCH:
            if n - GBUF >= 0:
                wh[n - GBUF].wait()
            gh[n] = start_gather(n)
    waited = set(n - GBUF for n in range(2, NGCH) if n - GBUF >= 0)
    for c in range(NGCH):
        if c not in waited:
            wh[c].wait()


def _gather(xt, src_token):
    mesh = plsc.VectorSubcoreMesh(core_axis_name="c", subcore_axis_name="s")
    kfn = functools.partial(
        pl.kernel, mesh=mesh,
        out_type=jax.ShapeDtypeStruct((P, D), jnp.float32),
        scratch_types=(
            [pltpu.VMEM((GCH,), jnp.int32) for _ in range(NGCH)]
            + [pltpu.VMEM((GCH, D), jnp.float32) for _ in range(GBUF)]
            + [pltpu.SemaphoreType.DMA]
            + [pltpu.SemaphoreType.DMA for _ in range(2 * GBUF)]
        ),
    )(_gather_body)
    return kfn(xt, src_token)


def _mm_body(bexp_ref, nblk_ref, xg_ref, we_ref, w_ref, yg_ref):
    i = pl.program_id(0)

    @pl.when(i < nblk_ref[0])
    def _():
        yg_ref[...] = lax.dot_general(
            xg_ref[...] * w_ref[...], we_ref[0],
            (((1,), (1,)), ((), ())),
            preferred_element_type=jnp.float32)


def _grouped_mm(xg, We, wsorted, bexp, nblk):
    grid_spec = pltpu.PrefetchScalarGridSpec(
        num_scalar_prefetch=2,
        grid=(NBLK,),
        in_specs=[
            pl.BlockSpec((BM, D), lambda i, bexp, nblk: (i, 0)),
            pl.BlockSpec((1, D, D), lambda i, bexp, nblk: (bexp[i], 0, 0)),
            pl.BlockSpec((BM, 1), lambda i, bexp, nblk: (i, 0)),
        ],
        out_specs=pl.BlockSpec((BM, D), lambda i, bexp, nblk: (i, 0)),
    )
    return pl.pallas_call(
        _mm_body,
        grid_spec=grid_spec,
        out_shape=jax.ShapeDtypeStruct((P, D), jnp.float32),
    )(bexp, nblk, xg, We, wsorted)


def _combine_body(yg_hbm, pos1_hbm, pos2_hbm, out_hbm, *rest):
    i1s = rest[:NCCH]
    i2s = rest[NCCH:2 * NCCH]
    r1, r2, isem, s1, s2, ws = rest[2 * NCCH:]
    wid = lax.axis_index("s") * NC + lax.axis_index("c")
    tbase = pl.multiple_of(wid * CTOK, CTOK)
    ih = ([pltpu.async_copy(pos1_hbm.at[pl.ds(tbase + c * CCH, CCH)], i1s[c], isem)
           for c in range(NCCH)]
          + [pltpu.async_copy(pos2_hbm.at[pl.ds(tbase + c * CCH, CCH)], i2s[c], isem)
             for c in range(NCCH)])
    for h in ih:
        h.wait()

    wh = None
    for c in range(NCCH):
        if wh is not None:
            wh.wait()
        h1 = pltpu.async_copy(yg_hbm.at[i1s[c]], r1, s1)
        h2 = pltpu.async_copy(yg_hbm.at[i2s[c]], r2, s2)
        h1.wait()
        h2.wait()

        def _row(i, _):
            for j in range(D // 16):
                v = r2[i, pl.ds(j * 16, 16)]
                plsc.addupdate(r1.at[i, pl.ds(j * 16, 16)], v)
            return 0

        lax.fori_loop(0, CCH, _row, 0)
        wh = pltpu.async_copy(r1, out_hbm.at[pl.ds(tbase + c * CCH, CCH)], ws)
    wh.wait()


def _combine(yg, pos1, pos2):
    mesh = plsc.VectorSubcoreMesh(core_axis_name="c", subcore_axis_name="s")
    kfn = functools.partial(
        pl.kernel, mesh=mesh,
        out_type=jax.ShapeDtypeStruct((T, D), jnp.float32),
        scratch_types=(
            [pltpu.VMEM((CCH,), jnp.int32) for _ in range(2 * NCCH)]
            + [pltpu.VMEM((CCH, D), jnp.float32) for _ in range(2)]
            + [pltpu.SemaphoreType.DMA for _ in range(4)]
        ),
    )(_combine_body)
    return kfn(yg, pos1, pos2)


def kernel(x, Wg, We):
    orig_shape = x.shape
    xt = x.reshape(-1, orig_shape[-1])
    comb = _gate(xt, Wg)
    src_token, wsorted, bexp, nblk, pos1, pos2 = _route(comb)
    xg = _gather(xt, src_token)
    yg = _grouped_mm(xg, We, wsorted, bexp, nblk)
    out = _combine(yg, pos1, pos2)
    return out.reshape(orig_shape)


# speedup vs baseline: 1.3792x; 1.3792x over previous
import functools

import jax
import jax.numpy as jnp
from jax import lax
from jax.experimental import pallas as pl
from jax.experimental.pallas import tpu as pltpu
from jax.experimental.pallas import tpu_sc as plsc

E = 8
D = 1024
T = 4096
A = 2 * T
BM = 256
NBLK = A // BM + E
P = NBLK * BM
NC = 2
NS = 16
NW = NC * NS

GROWS = P // NW
GCH = 40
NGCH = GROWS // GCH
GBUF = 3

CTOK = T // NW
CCH = 32
NCCH = CTOK // CCH


def _gate_body(x_ref, wg_ref, comb_ref):
    xblk = x_ref[...]
    logits = lax.dot_general(
        xblk, wg_ref[...], (((1,), (1,)), ((), ())),
        preferred_element_type=jnp.float32)
    z = logits - jnp.max(logits, axis=-1, keepdims=True)
    p = jnp.exp(z)
    p = p / jnp.sum(p, axis=-1, keepdims=True)
    idx = lax.broadcasted_iota(jnp.int32, p.shape, 1)
    m1 = jnp.max(p, axis=-1, keepdims=True)
    i1 = jnp.min(jnp.where(p == m1, idx, E), axis=-1, keepdims=True)
    sel1 = idx == i1
    pm = jnp.where(sel1, -1.0, p)
    m2 = jnp.max(pm, axis=-1, keepdims=True)
    i2 = jnp.min(jnp.where(pm == m2, idx, E), axis=-1, keepdims=True)
    sel2 = idx == i2
    denom = m1 + m2 + 1e-9
    comb_ref[...] = (jnp.where(sel1, m1 / denom, 0.0)
                     + jnp.where(sel2, m2 / denom, 0.0))


def _gate(xt, Wg):
    return pl.pallas_call(
        _gate_body,
        out_shape=jax.ShapeDtypeStruct((T, E), jnp.float32),
    )(xt, Wg)


def _route(comb):
    rows = jnp.arange(T)
    e1 = jnp.argmax(comb, axis=-1).astype(jnp.int32)
    w1 = jnp.max(comb, axis=-1)
    combm = comb.at[rows, e1].set(-1.0)
    e2 = jnp.argmax(combm, axis=-1).astype(jnp.int32)
    w2 = jnp.max(combm, axis=-1)

    ea = jnp.concatenate([e1, e2])
    wa = jnp.concatenate([w1, w2])
    ta = jnp.concatenate([rows, rows]).astype(jnp.int32)

    onehot = (ea[:, None] == jnp.arange(E)[None, :])
    cnt = jnp.sum(onehot, axis=0).astype(jnp.int32)
    pcnt = ((cnt + BM - 1) // BM) * BM
    pend = jnp.cumsum(pcnt)
    pstart = pend - pcnt
    cend = jnp.cumsum(cnt)
    cstart = cend - cnt

    order = jnp.argsort(ea, stable=True)
    inv = jnp.zeros((A,), jnp.int32).at[order].set(jnp.arange(A, dtype=jnp.int32))
    pos = inv - cstart[ea] + pstart[ea]

    src_token = (jnp.arange(P, dtype=jnp.int32) % T).at[pos].set(ta)
    wsorted = jnp.zeros((P,), jnp.float32).at[pos].set(wa)

    blk_base = jnp.arange(NBLK, dtype=jnp.int32) * BM
    bexp = jnp.sum(blk_base[:, None] >= pend[None, :], axis=-1).astype(jnp.int32)
    bexp = jnp.minimum(bexp, E - 1)
    nblk = (pend[-1] // BM).astype(jnp.int32).reshape((1,))
    return src_token, wsorted.reshape(P, 1), bexp, nblk, pos[:T], pos[T:]


def _gather_body(xt_hbm, src_hbm, xg_hbm, *rest):
    ics = rest[:NGCH]
    bufs = rest[NGCH:NGCH + GBUF]
    isem = rest[NGCH + GBUF]
    gsems = rest[NGCH + GBUF + 1:NGCH + 2 * GBUF + 1]
    wsems = rest[NGCH + 2 * GBUF + 1:NGCH + 3 * GBUF + 1]
    wid = lax.axis_index("s") * NC + lax.axis_index("c")
    base = pl.multiple_of(wid * GROWS, GROWS)
    ih = [pltpu.async_copy(src_hbm.at[pl.ds(base + c * GCH, GCH)], ics[c], isem)
          for c in range(NGCH)]
    for h in ih:
        h.wait()

    def start_gather(c):
        return pltpu.async_copy(xt_hbm.at[ics[c]], bufs[c % GBUF],
                                gsems[c % GBUF])

    gh = {}
    wh = {}
    for c in range(min(2, NGCH)):
        gh[c] = start_gather(c)
    for c in range(NGCH):
        gh[c].wait()
        wh[c] = pltpu.async_copy(
            bufs[c % GBUF], xg_hbm.at[pl.ds(base + c * GCH, GCH)],
            wsems[c % GBUF])
        n = c + 2
        if n < NGCH:
            if n - GBUF >= 0:
                wh[n - GBUF].wait()
            gh[n] = start_gather(n)
    waited = set(n - GBUF for n in range(2, NGCH) if n - GBUF >= 0)
    for c in range(NGCH):
        if c not in waited:
            wh[c].wait()


def _gather(xt, src_token):
    mesh = plsc.VectorSubcoreMesh(core_axis_name="c", subcore_axis_name="s")
    kfn = functools.partial(
        pl.kernel, mesh=mesh,
        out_type=jax.ShapeDtypeStruct((P, D), jnp.float32),
        scratch_types=(
            [pltpu.VMEM((GCH,), jnp.int32) for _ in range(NGCH)]
            + [pltpu.VMEM((GCH, D), jnp.float32) for _ in range(GBUF)]
            + [pltpu.SemaphoreType.DMA]
            + [pltpu.SemaphoreType.DMA for _ in range(2 * GBUF)]
        ),
    )(_gather_body)
    return kfn(xt, src_token)


def _mm_body(bexp_ref, nblk_ref, xg_ref, we_ref, w_ref, yg_ref):
    i = pl.program_id(0)

    @pl.when(i < nblk_ref[0])
    def _():
        yg_ref[...] = lax.dot_general(
            xg_ref[...] * w_ref[...], we_ref[0],
            (((1,), (1,)), ((), ())),
            preferred_element_type=jnp.float32)


def _grouped_mm(xg, We, wsorted, bexp, nblk):
    grid_spec = pltpu.PrefetchScalarGridSpec(
        num_scalar_prefetch=2,
        grid=(NBLK,),
        in_specs=[
            pl.BlockSpec((BM, D), lambda i, bexp, nblk: (i, 0)),
            pl.BlockSpec((1, D, D), lambda i, bexp, nblk: (bexp[i], 0, 0)),
            pl.BlockSpec((BM, 1), lambda i, bexp, nblk: (i, 0)),
        ],
        out_specs=pl.BlockSpec((BM, D), lambda i, bexp, nblk: (i, 0)),
    )
    return pl.pallas_call(
        _mm_body,
        grid_spec=grid_spec,
        out_shape=jax.ShapeDtypeStruct((P, D), jnp.float32),
    )(bexp, nblk, xg, We, wsorted)


def _combine_body(yg_hbm, pos1_hbm, pos2_hbm, out_hbm, *rest):
    i1s = rest[:NCCH]
    i2s = rest[NCCH:2 * NCCH]
    r1, r2, isem, s1, s2, ws = rest[2 * NCCH:]
    wid = lax.axis_index("s") * NC + lax.axis_index("c")
    tbase = pl.multiple_of(wid * CTOK, CTOK)
    ih = ([pltpu.async_copy(pos1_hbm.at[pl.ds(tbase + c * CCH, CCH)], i1s[c], isem)
           for c in range(NCCH)]
          + [pltpu.async_copy(pos2_hbm.at[pl.ds(tbase + c * CCH, CCH)], i2s[c], isem)
             for c in range(NCCH)])
    for h in ih:
        h.wait()

    wh = None
    for c in range(NCCH):
        if wh is not None:
            wh.wait()
        h1 = pltpu.async_copy(yg_hbm.at[i1s[c]], r1, s1)
        h2 = pltpu.async_copy(yg_hbm.at[i2s[c]], r2, s2)
        h1.wait()
        h2.wait()

        def _row(i, _):
            for j in range(D // 16):
                v = r2[i, pl.ds(j * 16, 16)]
                plsc.addupdate(r1.at[i, pl.ds(j * 16, 16)], v)
            return 0

        lax.fori_loop(0, CCH, _row, 0)
        wh = pltpu.async_copy(r1, out_hbm.at[pl.ds(tbase + c * CCH, CCH)], ws)
    wh.wait()


def _combine(yg, pos1, pos2):
    mesh = plsc.VectorSubcoreMesh(core_axis_name="c", subcore_axis_name="s")
    kfn = functools.partial(
        pl.kernel, mesh=mesh,
        out_type=jax.ShapeDtypeStruct((T, D), jnp.float32),
        scratch_types=(
            [pltpu.VMEM((CCH,), jnp.int32) for _ in range(2 * NCCH)]
            + [pltpu.VMEM((CCH, D), jnp.float32) for _ in range(2)]
            + [pltpu.SemaphoreType.DMA for _ in range(4)]
        ),
    )(_combine_body)
    return kfn(yg, pos1, pos2)


def kernel(x, Wg, We):
    orig_shape = x.shape
    xt = x.reshape(-1, orig_shape[-1])
    comb = _gate(xt, Wg)
    src_token, wsorted, bexp, nblk, pos1, pos2 = _route(comb)
    xg = _gather(xt, src_token)
    yg = _grouped_mm(xg, We, wsorted, bexp, nblk)
    out = _combine(yg, pos1, pos2)
    return out.reshape(orig_shape)


# speedup vs baseline: 3.7912x; 2.7488x over previous
import functools

import jax
import jax.numpy as jnp
from jax import lax
from jax.experimental import pallas as pl
from jax.experimental.pallas import tpu as pltpu

E = 8
D = 1024
TOPK = 2


def _moe_dense_body(x_ref, wg_ref, we_ref, out_ref, comb_ref):
    d = pl.program_id(0)
    e = pl.program_id(1)

    @pl.when((e == 0) & (d == 0))
    def _gate():
        xblk = x_ref[...]
        wg = wg_ref[...]
        logits = lax.dot_general(
            xblk, wg, (((1,), (1,)), ((), ())),
            preferred_element_type=jnp.float32)
        z = logits - jnp.max(logits, axis=-1, keepdims=True)
        p = jnp.exp(z)
        p = p / jnp.sum(p, axis=-1, keepdims=True)
        idx = lax.broadcasted_iota(jnp.int32, p.shape, 1)
        m1 = jnp.max(p, axis=-1, keepdims=True)
        i1 = jnp.min(jnp.where(p == m1, idx, E), axis=-1, keepdims=True)
        sel1 = idx == i1
        pm = jnp.where(sel1, -1.0, p)
        m2 = jnp.max(pm, axis=-1, keepdims=True)
        i2 = jnp.min(jnp.where(pm == m2, idx, E), axis=-1, keepdims=True)
        sel2 = idx == i2
        denom = m1 + m2 + 1e-9
        comb_ref[...] = (jnp.where(sel1, m1 / denom, 0.0)
                         + jnp.where(sel2, m2 / denom, 0.0))

    comb = comb_ref[...]
    eidx = lax.broadcasted_iota(jnp.int32, comb.shape, 1)
    scale = jnp.sum(jnp.where(eidx == e, comb, 0.0), axis=-1, keepdims=True)
    y = lax.dot_general(
        x_ref[...], we_ref[0], (((1,), (1,)), ((), ())),
        preferred_element_type=jnp.float32)

    @pl.when(e == 0)
    def _init():
        out_ref[...] = scale * y

    @pl.when(e > 0)
    def _accum():
        out_ref[...] += scale * y


def kernel(x, Wg, We):
    orig_shape = x.shape
    xt = x.reshape(-1, orig_shape[-1])
    T = xt.shape[0]
    BM = 4096
    BD = 512
    grid = (D // BD, E)
    out = pl.pallas_call(
        _moe_dense_body,
        grid=grid,
        in_specs=[
            pl.BlockSpec((BM, D), lambda d, e: (0, 0)),
            pl.BlockSpec((E, D), lambda d, e: (0, 0)),
            pl.BlockSpec((1, BD, D), lambda d, e: (e, d, 0)),
        ],
        out_specs=pl.BlockSpec((BM, BD), lambda d, e: (0, d)),
        out_shape=jax.ShapeDtypeStruct((T, D), jnp.float32),
        scratch_shapes=[
            pltpu.VMEM((BM, E), jnp.float32),
        ],
    )(xt, Wg, We)
    return out.reshape(orig_shape)


# speedup vs baseline: 3.8487x; 1.0152x over previous
import functools

import jax
import jax.numpy as jnp
from jax import lax
from jax.experimental import pallas as pl
from jax.experimental.pallas import tpu as pltpu

E = 8
D = 1024
TOPK = 2


def _moe_dense_body(x_ref, wg_ref, we_ref, out_ref, comb_ref):
    e = pl.program_id(1)

    @pl.when(e == 0)
    def _gate():
        xblk = x_ref[...]
        wg = wg_ref[...]
        logits = lax.dot_general(
            xblk, wg, (((1,), (1,)), ((), ())),
            preferred_element_type=jnp.float32)
        z = logits - jnp.max(logits, axis=-1, keepdims=True)
        p = jnp.exp(z)
        p = p / jnp.sum(p, axis=-1, keepdims=True)
        idx = lax.broadcasted_iota(jnp.int32, p.shape, 1)
        m1 = jnp.max(p, axis=-1, keepdims=True)
        i1 = jnp.min(jnp.where(p == m1, idx, E), axis=-1, keepdims=True)
        sel1 = idx == i1
        pm = jnp.where(sel1, -1.0, p)
        m2 = jnp.max(pm, axis=-1, keepdims=True)
        i2 = jnp.min(jnp.where(pm == m2, idx, E), axis=-1, keepdims=True)
        sel2 = idx == i2
        denom = m1 + m2 + 1e-9
        comb_ref[...] = (jnp.where(sel1, m1 / denom, 0.0)
                         + jnp.where(sel2, m2 / denom, 0.0))

    comb = comb_ref[...]
    eidx = lax.broadcasted_iota(jnp.int32, comb.shape, 1)
    scale = jnp.sum(jnp.where(eidx == e, comb, 0.0), axis=-1, keepdims=True)
    y = lax.dot_general(
        x_ref[...], we_ref[0], (((1,), (1,)), ((), ())),
        preferred_element_type=jnp.float32)

    @pl.when(e == 0)
    def _init():
        out_ref[...] = scale * y

    @pl.when(e > 0)
    def _accum():
        out_ref[...] += scale * y


def kernel(x, Wg, We):
    orig_shape = x.shape
    xt = x.reshape(-1, orig_shape[-1])
    T = xt.shape[0]
    BM = 2048
    BD = 1024
    grid = (T // BM, E)
    out = pl.pallas_call(
        _moe_dense_body,
        grid=grid,
        in_specs=[
            pl.BlockSpec((BM, D), lambda t, e: (t, 0)),
            pl.BlockSpec((E, D), lambda t, e: (0, 0)),
            pl.BlockSpec((1, BD, D), lambda t, e: (e, 0, 0)),
        ],
        out_specs=pl.BlockSpec((BM, BD), lambda t, e: (t, 0)),
        out_shape=jax.ShapeDtypeStruct((T, D), jnp.float32),
        scratch_shapes=[
            pltpu.VMEM((BM, E), jnp.float32),
        ],
    )(xt, Wg, We)
    return out.reshape(orig_shape)


# speedup vs baseline: 3.8670x; 1.0047x over previous
import jax
import jax.numpy as jnp
from jax import lax
from jax.experimental import pallas as pl
from jax.experimental.pallas import tpu as pltpu

E = 8
D = 1024
TOPK = 2


def _moe_dense_body(x_ref, wg_ref, we_ref, out_ref, comb_ref):
    e = pl.program_id(1)

    @pl.when(e == 0)
    def _gate():
        xblk = x_ref[...]
        wg = wg_ref[...]
        logits = lax.dot_general(
            xblk, wg, (((1,), (1,)), ((), ())),
            preferred_element_type=jnp.float32)
        z = logits - jnp.max(logits, axis=-1, keepdims=True)
        p = jnp.exp(z)
        p = p / jnp.sum(p, axis=-1, keepdims=True)
        idx = lax.broadcasted_iota(jnp.int32, p.shape, 1)
        m1 = jnp.max(p, axis=-1, keepdims=True)
        i1 = jnp.min(jnp.where(p == m1, idx, E), axis=-1, keepdims=True)
        sel1 = idx == i1
        pm = jnp.where(sel1, -1.0, p)
        m2 = jnp.max(pm, axis=-1, keepdims=True)
        i2 = jnp.min(jnp.where(pm == m2, idx, E), axis=-1, keepdims=True)
        sel2 = idx == i2
        denom = m1 + m2 + 1e-9
        comb_ref[...] = (jnp.where(sel1, m1 / denom, 0.0)
                         + jnp.where(sel2, m2 / denom, 0.0))

    comb = comb_ref[...]
    eidx = lax.broadcasted_iota(jnp.int32, comb.shape, 1)
    scale = jnp.sum(jnp.where(eidx == e, comb, 0.0), axis=-1, keepdims=True)
    y = lax.dot_general(
        x_ref[...], we_ref[0], (((1,), (1,)), ((), ())),
        preferred_element_type=jnp.float32)

    @pl.when(e == 0)
    def _init():
        out_ref[...] = scale * y

    @pl.when(e > 0)
    def _accum():
        out_ref[...] += scale * y


def kernel(x, Wg, We):
    orig_shape = x.shape
    xt = x.reshape(-1, orig_shape[-1])
    T = xt.shape[0]
    BM = 2048
    BD = 1024
    grid = (T // BM, E)
    out = pl.pallas_call(
        _moe_dense_body,
        grid=grid,
        in_specs=[
            pl.BlockSpec((BM, D), lambda t, e: (t, 0)),
            pl.BlockSpec((E, D), lambda t, e: (0, 0)),
            pl.BlockSpec((1, BD, D), lambda t, e: (e, 0, 0)),
        ],
        out_specs=pl.BlockSpec((BM, BD), lambda t, e: (t, 0)),
        out_shape=jax.ShapeDtypeStruct((T, D), jnp.float32),
        scratch_shapes=[
            pltpu.VMEM((BM, E), jnp.float32),
        ],
    )(xt, Wg, We)
    return out.reshape(orig_shape)
